# baseline (speedup 1.0000x reference)
import functools

import jax
import jax.numpy as jnp
from jax import lax
from jax.experimental import pallas as pl
from jax.experimental.pallas import tpu as pltpu
from jax.experimental.pallas import tpu_sc as plsc

N_SPARSE = 26
N_DENSE = 13
VOCAB = 100000
DIM = 32
B = 4096
OUT_W = N_SPARSE * DIM + N_DENSE

NC = 2
NS = 16
NW = NC * NS
BPW = B // NW
IPW = BPW * N_SPARSE


def _make_sc_embed():
    mesh = plsc.VectorSubcoreMesh(core_axis_name="c", subcore_axis_name="s")

    @functools.partial(
        pl.kernel,
        mesh=mesh,
        out_type=jax.ShapeDtypeStruct((B, OUT_W), jnp.float32),
        scratch_types=[
            pltpu.VMEM((IPW,), jnp.int32),
            pltpu.VMEM((IPW, DIM), jnp.float32),
            pltpu.VMEM((BPW, N_DENSE), jnp.float32),
            pltpu.SemaphoreType.DMA,
        ],
        compiler_params=pltpu.CompilerParams(use_tc_tiling_on_sc=False),
    )
    def sc_embed(dense_hbm, idx_hbm, tables_hbm, out_hbm,
                 idx_v, emb_v, dense_v, sem):
        wid = lax.axis_index("s") * NC + lax.axis_index("c")
        base = wid * BPW
        pltpu.sync_copy(idx_hbm.at[pl.ds(wid * IPW, IPW)], idx_v)
        pltpu.sync_copy(dense_hbm.at[pl.ds(base, BPW)], dense_v)
        copies = []
        for f in range(N_SPARSE):
            copies.append(pltpu.async_copy(
                tables_hbm.at[idx_v.at[pl.ds(f * BPW, BPW)]],
                emb_v.at[pl.ds(f * BPW, BPW)],
                sem))
        for cp in copies:
            cp.wait()
        for f in range(N_SPARSE):
            pltpu.sync_copy(
                emb_v.at[pl.ds(f * BPW, BPW)],
                out_hbm.at[pl.ds(base, BPW), pl.ds(f * DIM, DIM)])
        pltpu.sync_copy(
            dense_v, out_hbm.at[pl.ds(base, BPW), pl.ds(N_SPARSE * DIM, N_DENSE)])

    return sc_embed


def kernel(x_dense, x_sparse, tables):
    offs = jnp.arange(N_SPARSE, dtype=jnp.int32) * VOCAB
    idx_g = x_sparse + offs[None, :]
    idx_fm = idx_g.reshape(NW, BPW, N_SPARSE).transpose(0, 2, 1).reshape(-1)
    tables_flat = tables.reshape(N_SPARSE * VOCAB, DIM)
    return _make_sc_embed()(x_dense, idx_fm, tables_flat)
